# baseline (speedup 1.0000x reference)
import functools

import jax
import jax.numpy as jnp
from jax import lax
from jax.experimental import pallas as pl
from jax.experimental.pallas import tpu as pltpu
from jax.experimental.pallas import tpu_sc as plsc

NC = 2
NS = 16
LANES = 16
NW = NC * NS
CH = 64
N_PAD = 10240
D = 128
BLK = 512


def _mesh():
    return plsc.VectorSubcoreMesh(
        core_axis_name="c", subcore_axis_name="s", num_cores=NC, num_subcores=NS
    )


def _make_deg_kernel(epw):
    nvec = epw // LANES
    stripe = N_PAD // NS
    nred = stripe // LANES

    @functools.partial(
        pl.kernel,
        out_type=jax.ShapeDtypeStruct((NC, N_PAD), jnp.float32),
        mesh=_mesh(),
        compiler_params=pltpu.CompilerParams(needs_layout_passes=False),
        scratch_types=[
            pltpu.VMEM((epw,), jnp.int32),
            pltpu.VMEM((N_PAD,), jnp.float32),
            pltpu.VMEM((NS, stripe), jnp.float32),
            pltpu.VMEM((stripe,), jnp.float32),
            pltpu.VMEM_SHARED((NS, N_PAD), jnp.float32),
        ],
    )
    def deg_kernel(dst_hbm, out_hbm, idx_v, hist_v, red_v, res_v, shared):
        ci = lax.axis_index("c")
        si = lax.axis_index("s")
        w = ci * NS + si
        zeros = jnp.zeros((LANES,), jnp.float32)

        def zero_body(k, carry):
            hist_v[pl.ds(k * LANES, LANES)] = zeros
            return carry

        lax.fori_loop(0, N_PAD // LANES, zero_body, 0)
        pltpu.sync_copy(dst_hbm.at[w], idx_v)
        ones = jnp.ones((LANES,), jnp.float32)

        def hist_body(i, carry):
            v = idx_v[pl.ds(i * LANES, LANES)]
            plsc.addupdate_scatter(hist_v, [v], ones)
            return carry

        lax.fori_loop(0, nvec, hist_body, 0)
        pltpu.sync_copy(hist_v, shared.at[si])
        plsc.subcore_barrier()
        pltpu.sync_copy(shared.at[:, pl.ds(si * stripe, stripe)], red_v)

        def red_body(j, carry):
            acc = red_v[0, pl.ds(j * LANES, LANES)]
            for r in range(1, NS):
                acc = acc + red_v[r, pl.ds(j * LANES, LANES)]
            res_v[pl.ds(j * LANES, LANES)] = acc
            return carry

        lax.fori_loop(0, nred, red_body, 0)
        pltpu.sync_copy(res_v, out_hbm.at[ci, pl.ds(si * stripe, stripe)])

    return deg_kernel


NB = 2


def _make_agg_kernel(chunks):
    stripe = N_PAD // NS

    @functools.partial(
        pl.kernel,
        out_type=jax.ShapeDtypeStruct((NC, N_PAD, D), jnp.float32),
        mesh=_mesh(),
        compiler_params=pltpu.CompilerParams(needs_layout_passes=False),
        scratch_types=[
            pltpu.VMEM((chunks, CH), jnp.int32),
            pltpu.VMEM((chunks, CH), jnp.int32),
            pltpu.VMEM((CH, D), jnp.float32),
            pltpu.VMEM_SHARED((N_PAD, D), jnp.float32),
            pltpu.SemaphoreType.DMA,
            pltpu.SemaphoreType.DMA,
        ],
    )
    def agg_kernel(y_hbm, src_hbm, dst_hbm, zero_hbm, out_hbm,
                   src_v, dst_v, buf_v, acc_sh, gsem, ssem):
        ci = lax.axis_index("c")
        si = lax.axis_index("s")
        w = ci * NS + si
        pltpu.sync_copy(zero_hbm.at[pl.ds(si * stripe, stripe)],
                        acc_sh.at[pl.ds(si * stripe, stripe)])
        pltpu.sync_copy(src_hbm.at[w], src_v)
        pltpu.sync_copy(dst_hbm.at[w], dst_v)
        plsc.subcore_barrier()

        def body(j, carry):
            pltpu.async_copy(y_hbm.at[src_v.at[j]], buf_v, gsem).wait()
            pltpu.async_copy(buf_v, acc_sh.at[dst_v.at[j]], ssem,
                             add=True).wait()
            return carry

        lax.fori_loop(0, chunks, body, 0)
        plsc.subcore_barrier()
        pltpu.sync_copy(acc_sh.at[pl.ds(si * stripe, stripe)],
                        out_hbm.at[ci, pl.ds(si * stripe, stripe)])

    return agg_kernel


def _tc_scale_matmul(xp, W, dis2):
    def body(x_ref, w_ref, d_ref, y_ref):
        y_ref[...] = d_ref[...] * jnp.dot(
            x_ref[...], w_ref[...], preferred_element_type=jnp.float32)

    return pl.pallas_call(
        body,
        grid=(N_PAD // BLK,),
        in_specs=[
            pl.BlockSpec((BLK, D), lambda i: (i, 0)),
            pl.BlockSpec((D, D), lambda i: (0, 0)),
            pl.BlockSpec((BLK, 1), lambda i: (i, 0)),
        ],
        out_specs=pl.BlockSpec((BLK, D), lambda i: (i, 0)),
        out_shape=jax.ShapeDtypeStruct((N_PAD, D), jnp.float32),
    )(xp, W, dis2)


def _tc_mid(p, y1, dis2, b1, W2, nvalid):
    def body(p_ref, y1_ref, d_ref, b_ref, w_ref, o_ref):
        i = pl.program_id(0)
        s = p_ref[0] + p_ref[1] + y1_ref[...]
        h = jnp.maximum(d_ref[...] * s + b_ref[...], 0.0)
        rows = i * BLK + lax.broadcasted_iota(jnp.int32, (BLK, 1), 0)
        h = jnp.where(rows < nvalid, h, 0.0)
        o_ref[...] = d_ref[...] * jnp.dot(
            h, w_ref[...], preferred_element_type=jnp.float32)

    return pl.pallas_call(
        body,
        grid=(N_PAD // BLK,),
        in_specs=[
            pl.BlockSpec((NC, BLK, D), lambda i: (0, i, 0)),
            pl.BlockSpec((BLK, D), lambda i: (i, 0)),
            pl.BlockSpec((BLK, 1), lambda i: (i, 0)),
            pl.BlockSpec((1, D), lambda i: (0, 0)),
            pl.BlockSpec((D, D), lambda i: (0, 0)),
        ],
        out_specs=pl.BlockSpec((BLK, D), lambda i: (i, 0)),
        out_shape=jax.ShapeDtypeStruct((N_PAD, D), jnp.float32),
    )(p, y1, dis2, b1, W2)


def _tc_final(q, y2, dis2, b2):
    def body(q_ref, y2_ref, d_ref, b_ref, o_ref):
        o_ref[...] = d_ref[...] * (q_ref[0] + q_ref[1] + y2_ref[...]) + b_ref[...]

    return pl.pallas_call(
        body,
        grid=(N_PAD // BLK,),
        in_specs=[
            pl.BlockSpec((NC, BLK, D), lambda i: (0, i, 0)),
            pl.BlockSpec((BLK, D), lambda i: (i, 0)),
            pl.BlockSpec((BLK, 1), lambda i: (i, 0)),
            pl.BlockSpec((1, D), lambda i: (0, 0)),
        ],
        out_specs=pl.BlockSpec((BLK, D), lambda i: (i, 0)),
        out_shape=jax.ShapeDtypeStruct((N_PAD, D), jnp.float32),
    )(q, y2, dis2, b2)


def kernel(x, edge_index, W1, b1, W2, b2):
    N, _ = x.shape
    E = edge_index.shape[1]
    src = edge_index[0].astype(jnp.int32)
    dst = edge_index[1].astype(jnp.int32)

    epw_raw = -(-E // NW)
    chunks = -(-epw_raw // CH)
    chunks = -(-chunks // (2 * NB)) * (2 * NB)
    epw = chunks * CH
    pad = epw * NW - E
    dum = N + (jnp.arange(pad, dtype=jnp.int32) % (N_PAD - N))
    srcp = jnp.concatenate([src, jnp.full((pad,), N, jnp.int32)]).reshape(NW, chunks, CH)
    dstp = jnp.concatenate([dst, dum]).reshape(NW, chunks, CH)
    xp = jnp.pad(x, ((0, N_PAD - N), (0, 0)))
    zeros = jnp.zeros((N_PAD, D), jnp.float32)

    degp = _make_deg_kernel(epw)(dstp.reshape(NW, epw))
    deg = degp[0] + degp[1] + 1.0
    dis2 = lax.rsqrt(deg)[:, None]

    agg = _make_agg_kernel(chunks)
    y1 = _tc_scale_matmul(xp, W1, dis2)
    p = agg(y1, srcp, dstp, zeros)
    y2 = _tc_mid(p, y1, dis2, b1.reshape(1, D), W2, N)
    q = agg(y2, srcp, dstp, zeros)
    out = _tc_final(q, y2, dis2, b2.reshape(1, D))
    return out[:N]


# speedup vs baseline: 2.3008x; 2.3008x over previous
import functools

import jax
import jax.numpy as jnp
from jax import lax
from jax.experimental import pallas as pl
from jax.experimental.pallas import tpu as pltpu
from jax.experimental.pallas import tpu_sc as plsc

NC = 2
NS = 16
LANES = 16
NW = NC * NS
CH = 64
N_PAD = 10240
D = 128
BLK = 512


def _mesh():
    return plsc.VectorSubcoreMesh(
        core_axis_name="c", subcore_axis_name="s", num_cores=NC, num_subcores=NS
    )


def _make_deg_kernel(epw):
    nvec = epw // LANES
    stripe = N_PAD // NS
    nred = stripe // LANES

    @functools.partial(
        pl.kernel,
        out_type=jax.ShapeDtypeStruct((NC, N_PAD), jnp.float32),
        mesh=_mesh(),
        compiler_params=pltpu.CompilerParams(needs_layout_passes=False),
        scratch_types=[
            pltpu.VMEM((epw,), jnp.int32),
            pltpu.VMEM((N_PAD,), jnp.float32),
            pltpu.VMEM((NS, stripe), jnp.float32),
            pltpu.VMEM((stripe,), jnp.float32),
            pltpu.VMEM_SHARED((NS, N_PAD), jnp.float32),
        ],
    )
    def deg_kernel(dst_hbm, out_hbm, idx_v, hist_v, red_v, res_v, shared):
        ci = lax.axis_index("c")
        si = lax.axis_index("s")
        w = ci * NS + si
        zeros = jnp.zeros((LANES,), jnp.float32)

        def zero_body(k, carry):
            hist_v[pl.ds(k * LANES, LANES)] = zeros
            return carry

        lax.fori_loop(0, N_PAD // LANES, zero_body, 0)
        pltpu.sync_copy(dst_hbm.at[w], idx_v)
        ones = jnp.ones((LANES,), jnp.float32)

        def hist_body(i, carry):
            v = idx_v[pl.ds(i * LANES, LANES)]
            plsc.addupdate_scatter(hist_v, [v], ones)
            return carry

        lax.fori_loop(0, nvec, hist_body, 0)
        pltpu.sync_copy(hist_v, shared.at[si])
        plsc.subcore_barrier()
        pltpu.sync_copy(shared.at[:, pl.ds(si * stripe, stripe)], red_v)

        def red_body(j, carry):
            acc = red_v[0, pl.ds(j * LANES, LANES)]
            for r in range(1, NS):
                acc = acc + red_v[r, pl.ds(j * LANES, LANES)]
            res_v[pl.ds(j * LANES, LANES)] = acc
            return carry

        lax.fori_loop(0, nred, red_body, 0)
        pltpu.sync_copy(res_v, out_hbm.at[ci, pl.ds(si * stripe, stripe)])

    return deg_kernel


NB = 2


def _make_agg_kernel(chunks):
    stripe = N_PAD // NS

    @functools.partial(
        pl.kernel,
        out_type=jax.ShapeDtypeStruct((NC, N_PAD, D), jnp.float32),
        mesh=_mesh(),
        compiler_params=pltpu.CompilerParams(needs_layout_passes=False),
        scratch_types=[
            pltpu.VMEM((chunks, CH), jnp.int32),
            pltpu.VMEM((chunks, CH), jnp.int32),
            pltpu.VMEM((CH, D), jnp.float32),
            pltpu.VMEM_SHARED((N_PAD, D), jnp.float32),
            pltpu.SemaphoreType.DMA,
            pltpu.SemaphoreType.DMA,
        ],
    )
    def agg_kernel(y_hbm, src_hbm, dst_hbm, zero_hbm, out_hbm,
                   src_v, dst_v, buf_v, acc_sh, gsem, ssem):
        ci = lax.axis_index("c")
        si = lax.axis_index("s")
        w = ci * NS + si
        pltpu.sync_copy(zero_hbm.at[pl.ds(si * stripe, stripe)],
                        acc_sh.at[pl.ds(si * stripe, stripe)])
        pltpu.sync_copy(src_hbm.at[w], src_v)
        pltpu.sync_copy(dst_hbm.at[w], dst_v)
        plsc.subcore_barrier()

        def body(j, carry):
            pltpu.async_copy(y_hbm.at[src_v.at[j]], buf_v, gsem).wait()
            pltpu.async_copy(buf_v, acc_sh.at[dst_v.at[j]], ssem,
                             add=True).wait()
            return carry

        lax.fori_loop(0, chunks, body, 0)
        plsc.subcore_barrier()
        pltpu.sync_copy(acc_sh.at[pl.ds(si * stripe, stripe)],
                        out_hbm.at[ci, pl.ds(si * stripe, stripe)])

    return agg_kernel


def _tc_scale_matmul(xp, W, dis2):
    def body(x_ref, w_ref, d_ref, y_ref):
        y_ref[...] = d_ref[...] * jnp.dot(
            x_ref[...], w_ref[...], preferred_element_type=jnp.float32)

    return pl.pallas_call(
        body,
        grid=(N_PAD // BLK,),
        in_specs=[
            pl.BlockSpec((BLK, D), lambda i: (i, 0)),
            pl.BlockSpec((D, D), lambda i: (0, 0)),
            pl.BlockSpec((BLK, 1), lambda i: (i, 0)),
        ],
        out_specs=pl.BlockSpec((BLK, D), lambda i: (i, 0)),
        out_shape=jax.ShapeDtypeStruct((N_PAD, D), jnp.float32),
    )(xp, W, dis2)


def _tc_mid(p, y1, dis2, b1, W2, nvalid):
    def body(p_ref, y1_ref, d_ref, b_ref, w_ref, o_ref):
        i = pl.program_id(0)
        s = p_ref[0] + p_ref[1] + y1_ref[...]
        h = jnp.maximum(d_ref[...] * s + b_ref[...], 0.0)
        rows = i * BLK + lax.broadcasted_iota(jnp.int32, (BLK, 1), 0)
        h = jnp.where(rows < nvalid, h, 0.0)
        o_ref[...] = d_ref[...] * jnp.dot(
            h, w_ref[...], preferred_element_type=jnp.float32)

    return pl.pallas_call(
        body,
        grid=(N_PAD // BLK,),
        in_specs=[
            pl.BlockSpec((NC, BLK, D), lambda i: (0, i, 0)),
            pl.BlockSpec((BLK, D), lambda i: (i, 0)),
            pl.BlockSpec((BLK, 1), lambda i: (i, 0)),
            pl.BlockSpec((1, D), lambda i: (0, 0)),
            pl.BlockSpec((D, D), lambda i: (0, 0)),
        ],
        out_specs=pl.BlockSpec((BLK, D), lambda i: (i, 0)),
        out_shape=jax.ShapeDtypeStruct((N_PAD, D), jnp.float32),
    )(p, y1, dis2, b1, W2)


def _tc_final(q, y2, dis2, b2):
    def body(q_ref, y2_ref, d_ref, b_ref, o_ref):
        o_ref[...] = d_ref[...] * (q_ref[0] + q_ref[1] + y2_ref[...]) + b_ref[...]

    return pl.pallas_call(
        body,
        grid=(N_PAD // BLK,),
        in_specs=[
            pl.BlockSpec((NC, BLK, D), lambda i: (0, i, 0)),
            pl.BlockSpec((BLK, D), lambda i: (i, 0)),
            pl.BlockSpec((BLK, 1), lambda i: (i, 0)),
            pl.BlockSpec((1, D), lambda i: (0, 0)),
        ],
        out_specs=pl.BlockSpec((BLK, D), lambda i: (i, 0)),
        out_shape=jax.ShapeDtypeStruct((N_PAD, D), jnp.float32),
    )(q, y2, dis2, b2)


def kernel(x, edge_index, W1, b1, W2, b2):
    N, _ = x.shape
    E = edge_index.shape[1]
    src = edge_index[0].astype(jnp.int32)
    dst = edge_index[1].astype(jnp.int32)

    epw_raw = -(-E // NW)
    chunks = -(-epw_raw // CH)
    epw = chunks * CH
    pad = epw * NW - E
    dum = N + (jnp.arange(pad, dtype=jnp.int32) % (N_PAD - N))
    srcp = jnp.concatenate([src, dum]).reshape(NW, chunks, CH)
    dstp = jnp.concatenate([dst, dum]).reshape(NW, chunks, CH)
    xp = jnp.pad(x, ((0, N_PAD - N), (0, 0)))
    zeros = jnp.zeros((N_PAD, D), jnp.float32)

    degp = _make_deg_kernel(epw)(dstp.reshape(NW, epw))
    deg = degp[0] + degp[1] + 1.0
    dis2 = lax.rsqrt(deg)[:, None]

    agg = _make_agg_kernel(chunks)
    y1 = _tc_scale_matmul(xp, W1, dis2)
    p = agg(y1, srcp, dstp, zeros)
    y2 = _tc_mid(p, y1, dis2, b1.reshape(1, D), W2, N)
    q = agg(y2, srcp, dstp, zeros)
    out = _tc_final(q, y2, dis2, b2.reshape(1, D))
    return out[:N]


# speedup vs baseline: 2.8192x; 1.2253x over previous
import functools

import jax
import jax.numpy as jnp
from jax import lax
from jax.experimental import pallas as pl
from jax.experimental.pallas import tpu as pltpu
from jax.experimental.pallas import tpu_sc as plsc

NC = 2
NS = 16
LANES = 16
NW = NC * NS
CH = 64
N_PAD = 10240
D = 128
BLK = 512


def _mesh():
    return plsc.VectorSubcoreMesh(
        core_axis_name="c", subcore_axis_name="s", num_cores=NC, num_subcores=NS
    )


def _make_deg_kernel(epw):
    nvec = epw // LANES
    stripe = N_PAD // NS
    nred = stripe // LANES

    @functools.partial(
        pl.kernel,
        out_type=jax.ShapeDtypeStruct((NC, N_PAD), jnp.float32),
        mesh=_mesh(),
        compiler_params=pltpu.CompilerParams(needs_layout_passes=False),
        scratch_types=[
            pltpu.VMEM((epw,), jnp.int32),
            pltpu.VMEM((N_PAD,), jnp.float32),
            pltpu.VMEM((NS, stripe), jnp.float32),
            pltpu.VMEM((stripe,), jnp.float32),
            pltpu.VMEM_SHARED((NS, N_PAD), jnp.float32),
        ],
    )
    def deg_kernel(dst_hbm, out_hbm, idx_v, hist_v, red_v, res_v, shared):
        ci = lax.axis_index("c")
        si = lax.axis_index("s")
        w = ci * NS + si
        zeros = jnp.zeros((LANES,), jnp.float32)

        def zero_body(k, carry):
            hist_v[pl.ds(k * LANES, LANES)] = zeros
            return carry

        lax.fori_loop(0, N_PAD // LANES, zero_body, 0)
        pltpu.sync_copy(dst_hbm.at[w], idx_v)
        ones = jnp.ones((LANES,), jnp.float32)

        def hist_body(i, carry):
            v = idx_v[pl.ds(i * LANES, LANES)]
            plsc.addupdate_scatter(hist_v, [v], ones)
            return carry

        lax.fori_loop(0, nvec, hist_body, 0)
        pltpu.sync_copy(hist_v, shared.at[si])
        plsc.subcore_barrier()
        pltpu.sync_copy(shared.at[:, pl.ds(si * stripe, stripe)], red_v)

        def red_body(j, carry):
            acc = red_v[0, pl.ds(j * LANES, LANES)]
            for r in range(1, NS):
                acc = acc + red_v[r, pl.ds(j * LANES, LANES)]
            res_v[pl.ds(j * LANES, LANES)] = acc
            return carry

        lax.fori_loop(0, nred, red_body, 0)
        pltpu.sync_copy(res_v, out_hbm.at[ci, pl.ds(si * stripe, stripe)])

    return deg_kernel


NB = 2


def _make_agg_kernel(chunks):
    stripe = N_PAD // NS
    epw = chunks * CH

    @functools.partial(
        pl.kernel,
        out_type=jax.ShapeDtypeStruct((NC, N_PAD, D), jnp.float32),
        mesh=_mesh(),
        compiler_params=pltpu.CompilerParams(needs_layout_passes=False),
        scratch_types=[
            pltpu.VMEM((epw,), jnp.int32),
            pltpu.VMEM((chunks, CH), jnp.int32),
            pltpu.VMEM((2, CH, D), jnp.float32),
            pltpu.VMEM_SHARED((N_PAD, D), jnp.float32),
        ]
        + [pltpu.SemaphoreType.DMA] * 4,
    )
    def agg_kernel(y_hbm, src_hbm, dst_hbm, zero_hbm, out_hbm,
                   src_v, dst_v, buf_v, acc_sh, gsem0, gsem1, ssem0, ssem1):
        gsems = (gsem0, gsem1)
        ssems = (ssem0, ssem1)
        ci = lax.axis_index("c")
        si = lax.axis_index("s")
        w = ci * NS + si
        pltpu.sync_copy(zero_hbm.at[pl.ds(si * stripe, stripe)],
                        acc_sh.at[pl.ds(si * stripe, stripe)])
        pltpu.sync_copy(src_hbm.at[w], src_v)
        pltpu.sync_copy(dst_hbm.at[w], dst_v)
        plsc.subcore_barrier()

        def gather(c, b):
            pltpu.async_copy(y_hbm.at[src_v.at[pl.ds(c * CH, CH)]],
                             buf_v.at[b], gsems[b])

        def gwait(b):
            pltpu.make_async_copy(y_hbm.at[src_v.at[pl.ds(0, CH)]],
                                  buf_v.at[b], gsems[b]).wait()

        def scat(c, b):
            pltpu.async_copy(buf_v.at[b], acc_sh.at[dst_v.at[c]], ssems[b],
                             add=True)

        def swait(b):
            pltpu.make_async_copy(buf_v.at[b], acc_sh.at[dst_v.at[0]],
                                  ssems[b]).wait()

        gather(0, 0)
        gwait(0)
        scat(0, 0)
        gather(1, 1)
        gwait(1)
        scat(1, 1)
        swait(0)
        gather(2, 0)

        def body(t, carry):
            c0 = 2 * t
            gwait(0)
            scat(c0, 0)
            swait(1)
            gather(c0 + 1, 1)
            gwait(1)
            scat(c0 + 1, 1)
            swait(0)
            gather(jnp.minimum(c0 + 2, chunks - 1), 0)
            return carry

        lax.fori_loop(1, chunks // 2, body, 0)
        gwait(0)
        swait(1)
        plsc.subcore_barrier()
        pltpu.sync_copy(acc_sh.at[pl.ds(si * stripe, stripe)],
                        out_hbm.at[ci, pl.ds(si * stripe, stripe)])

    return agg_kernel


def _tc_scale_matmul(xp, W, dis2):
    def body(x_ref, w_ref, d_ref, y_ref):
        y_ref[...] = d_ref[...] * jnp.dot(
            x_ref[...], w_ref[...], preferred_element_type=jnp.float32)

    return pl.pallas_call(
        body,
        grid=(N_PAD // BLK,),
        in_specs=[
            pl.BlockSpec((BLK, D), lambda i: (i, 0)),
            pl.BlockSpec((D, D), lambda i: (0, 0)),
            pl.BlockSpec((BLK, 1), lambda i: (i, 0)),
        ],
        out_specs=pl.BlockSpec((BLK, D), lambda i: (i, 0)),
        out_shape=jax.ShapeDtypeStruct((N_PAD, D), jnp.float32),
    )(xp, W, dis2)


def _tc_mid(p, y1, dis2, b1, W2, nvalid):
    def body(p_ref, y1_ref, d_ref, b_ref, w_ref, o_ref):
        i = pl.program_id(0)
        s = p_ref[0] + p_ref[1] + y1_ref[...]
        h = jnp.maximum(d_ref[...] * s + b_ref[...], 0.0)
        rows = i * BLK + lax.broadcasted_iota(jnp.int32, (BLK, 1), 0)
        h = jnp.where(rows < nvalid, h, 0.0)
        o_ref[...] = d_ref[...] * jnp.dot(
            h, w_ref[...], preferred_element_type=jnp.float32)

    return pl.pallas_call(
        body,
        grid=(N_PAD // BLK,),
        in_specs=[
            pl.BlockSpec((NC, BLK, D), lambda i: (0, i, 0)),
            pl.BlockSpec((BLK, D), lambda i: (i, 0)),
            pl.BlockSpec((BLK, 1), lambda i: (i, 0)),
            pl.BlockSpec((1, D), lambda i: (0, 0)),
            pl.BlockSpec((D, D), lambda i: (0, 0)),
        ],
        out_specs=pl.BlockSpec((BLK, D), lambda i: (i, 0)),
        out_shape=jax.ShapeDtypeStruct((N_PAD, D), jnp.float32),
    )(p, y1, dis2, b1, W2)


def _tc_final(q, y2, dis2, b2):
    def body(q_ref, y2_ref, d_ref, b_ref, o_ref):
        o_ref[...] = d_ref[...] * (q_ref[0] + q_ref[1] + y2_ref[...]) + b_ref[...]

    return pl.pallas_call(
        body,
        grid=(N_PAD // BLK,),
        in_specs=[
            pl.BlockSpec((NC, BLK, D), lambda i: (0, i, 0)),
            pl.BlockSpec((BLK, D), lambda i: (i, 0)),
            pl.BlockSpec((BLK, 1), lambda i: (i, 0)),
            pl.BlockSpec((1, D), lambda i: (0, 0)),
        ],
        out_specs=pl.BlockSpec((BLK, D), lambda i: (i, 0)),
        out_shape=jax.ShapeDtypeStruct((N_PAD, D), jnp.float32),
    )(q, y2, dis2, b2)


def kernel(x, edge_index, W1, b1, W2, b2):
    N, _ = x.shape
    E = edge_index.shape[1]
    src = edge_index[0].astype(jnp.int32)
    dst = edge_index[1].astype(jnp.int32)

    epw_raw = -(-E // NW)
    chunks = -(-epw_raw // CH)
    chunks += chunks % 2
    epw = chunks * CH
    pad = epw * NW - E
    dum = N + (jnp.arange(pad, dtype=jnp.int32) % (N_PAD - N))
    srcp = jnp.concatenate([src, dum]).reshape(NW, chunks, CH)
    dstp = jnp.concatenate([dst, dum]).reshape(NW, chunks, CH)
    xp = jnp.pad(x, ((0, N_PAD - N), (0, 0)))
    zeros = jnp.zeros((N_PAD, D), jnp.float32)

    degp = _make_deg_kernel(epw)(dstp.reshape(NW, epw))
    deg = degp[0] + degp[1] + 1.0
    dis2 = lax.rsqrt(deg)[:, None]

    agg = _make_agg_kernel(chunks)
    y1 = _tc_scale_matmul(xp, W1, dis2)
    src1d = srcp.reshape(NW, epw)
    p = agg(y1, src1d, dstp, zeros)
    y2 = _tc_mid(p, y1, dis2, b1.reshape(1, D), W2, N)
    q = agg(y2, src1d, dstp, zeros)
    out = _tc_final(q, y2, dis2, b2.reshape(1, D))
    return out[:N]


# speedup vs baseline: 3.2549x; 1.1545x over previous
import functools

import jax
import jax.numpy as jnp
from jax import lax
from jax.experimental import pallas as pl
from jax.experimental.pallas import tpu as pltpu
from jax.experimental.pallas import tpu_sc as plsc

NC = 2
NS = 16
LANES = 16
NW = NC * NS
CH = 96
N_PAD = 10240
D = 128
BLK = 512


def _mesh():
    return plsc.VectorSubcoreMesh(
        core_axis_name="c", subcore_axis_name="s", num_cores=NC, num_subcores=NS
    )


def _make_deg_kernel(epw):
    nvec = epw // LANES
    stripe = N_PAD // NS
    nred = stripe // LANES

    @functools.partial(
        pl.kernel,
        out_type=jax.ShapeDtypeStruct((NC, N_PAD), jnp.float32),
        mesh=_mesh(),
        compiler_params=pltpu.CompilerParams(needs_layout_passes=False),
        scratch_types=[
            pltpu.VMEM((epw,), jnp.int32),
            pltpu.VMEM((N_PAD,), jnp.float32),
            pltpu.VMEM((NS, stripe), jnp.float32),
            pltpu.VMEM((stripe,), jnp.float32),
            pltpu.VMEM_SHARED((NS, N_PAD), jnp.float32),
        ],
    )
    def deg_kernel(dst_hbm, out_hbm, idx_v, hist_v, red_v, res_v, shared):
        ci = lax.axis_index("c")
        si = lax.axis_index("s")
        w = ci * NS + si
        zeros = jnp.zeros((LANES,), jnp.float32)

        def zero_body(k, carry):
            hist_v[pl.ds(k * LANES, LANES)] = zeros
            return carry

        lax.fori_loop(0, N_PAD // LANES, zero_body, 0)
        pltpu.sync_copy(dst_hbm.at[w], idx_v)
        ones = jnp.ones((LANES,), jnp.float32)

        def hist_body(i, carry):
            v = idx_v[pl.ds(i * LANES, LANES)]
            plsc.addupdate_scatter(hist_v, [v], ones)
            return carry

        lax.fori_loop(0, nvec, hist_body, 0)
        pltpu.sync_copy(hist_v, shared.at[si])
        plsc.subcore_barrier()
        pltpu.sync_copy(shared.at[:, pl.ds(si * stripe, stripe)], red_v)

        def red_body(j, carry):
            acc = red_v[0, pl.ds(j * LANES, LANES)]
            for r in range(1, NS):
                acc = acc + red_v[r, pl.ds(j * LANES, LANES)]
            res_v[pl.ds(j * LANES, LANES)] = acc
            return carry

        lax.fori_loop(0, nred, red_body, 0)
        pltpu.sync_copy(res_v, out_hbm.at[ci, pl.ds(si * stripe, stripe)])

    return deg_kernel


NB = 2


def _make_agg_kernel(chunks):
    stripe = N_PAD // NS
    epw = chunks * CH

    @functools.partial(
        pl.kernel,
        out_type=jax.ShapeDtypeStruct((NC, N_PAD, D), jnp.float32),
        mesh=_mesh(),
        compiler_params=pltpu.CompilerParams(needs_layout_passes=False),
        scratch_types=[
            pltpu.VMEM((epw,), jnp.int32),
            pltpu.VMEM((chunks, CH), jnp.int32),
            pltpu.VMEM((2, CH, D), jnp.float32),
            pltpu.VMEM_SHARED((N_PAD, D), jnp.float32),
        ]
        + [pltpu.SemaphoreType.DMA] * 4,
    )
    def agg_kernel(y_hbm, src_hbm, dst_hbm, zero_hbm, out_hbm,
                   src_v, dst_v, buf_v, acc_sh, gsem0, gsem1, ssem0, ssem1):
        gsems = (gsem0, gsem1)
        ssems = (ssem0, ssem1)
        ci = lax.axis_index("c")
        si = lax.axis_index("s")
        w = ci * NS + si
        pltpu.sync_copy(zero_hbm.at[pl.ds(si * stripe, stripe)],
                        acc_sh.at[pl.ds(si * stripe, stripe)])
        pltpu.sync_copy(src_hbm.at[w], src_v)
        pltpu.sync_copy(dst_hbm.at[w], dst_v)
        plsc.subcore_barrier()

        def gather(c, b):
            pltpu.async_copy(y_hbm.at[src_v.at[pl.ds(c * CH, CH)]],
                             buf_v.at[b], gsems[b])

        def gwait(b):
            pltpu.make_async_copy(y_hbm.at[src_v.at[pl.ds(0, CH)]],
                                  buf_v.at[b], gsems[b]).wait()

        def scat(c, b):
            pltpu.async_copy(buf_v.at[b], acc_sh.at[dst_v.at[c]], ssems[b],
                             add=True)

        def swait(b):
            pltpu.make_async_copy(buf_v.at[b], acc_sh.at[dst_v.at[0]],
                                  ssems[b]).wait()

        gather(0, 0)
        gwait(0)
        scat(0, 0)
        gather(1, 1)
        gwait(1)
        scat(1, 1)
        swait(0)
        gather(2, 0)

        def body(t, carry):
            c0 = 2 * t
            gwait(0)
            scat(c0, 0)
            swait(1)
            gather(c0 + 1, 1)
            gwait(1)
            scat(c0 + 1, 1)
            swait(0)
            gather(jnp.minimum(c0 + 2, chunks - 1), 0)
            return carry

        lax.fori_loop(1, chunks // 2, body, 0)
        gwait(0)
        swait(1)
        plsc.subcore_barrier()
        pltpu.sync_copy(acc_sh.at[pl.ds(si * stripe, stripe)],
                        out_hbm.at[ci, pl.ds(si * stripe, stripe)])

    return agg_kernel


def _tc_scale_matmul(xp, W, dis2):
    def body(x_ref, w_ref, d_ref, y_ref):
        y_ref[...] = d_ref[...] * jnp.dot(
            x_ref[...], w_ref[...], preferred_element_type=jnp.float32)

    return pl.pallas_call(
        body,
        grid=(N_PAD // BLK,),
        in_specs=[
            pl.BlockSpec((BLK, D), lambda i: (i, 0)),
            pl.BlockSpec((D, D), lambda i: (0, 0)),
            pl.BlockSpec((BLK, 1), lambda i: (i, 0)),
        ],
        out_specs=pl.BlockSpec((BLK, D), lambda i: (i, 0)),
        out_shape=jax.ShapeDtypeStruct((N_PAD, D), jnp.float32),
    )(xp, W, dis2)


def _tc_mid(p, y1, dis2, b1, W2, nvalid):
    def body(p_ref, y1_ref, d_ref, b_ref, w_ref, o_ref):
        i = pl.program_id(0)
        s = p_ref[0] + p_ref[1] + y1_ref[...]
        h = jnp.maximum(d_ref[...] * s + b_ref[...], 0.0)
        rows = i * BLK + lax.broadcasted_iota(jnp.int32, (BLK, 1), 0)
        h = jnp.where(rows < nvalid, h, 0.0)
        o_ref[...] = d_ref[...] * jnp.dot(
            h, w_ref[...], preferred_element_type=jnp.float32)

    return pl.pallas_call(
        body,
        grid=(N_PAD // BLK,),
        in_specs=[
            pl.BlockSpec((NC, BLK, D), lambda i: (0, i, 0)),
            pl.BlockSpec((BLK, D), lambda i: (i, 0)),
            pl.BlockSpec((BLK, 1), lambda i: (i, 0)),
            pl.BlockSpec((1, D), lambda i: (0, 0)),
            pl.BlockSpec((D, D), lambda i: (0, 0)),
        ],
        out_specs=pl.BlockSpec((BLK, D), lambda i: (i, 0)),
        out_shape=jax.ShapeDtypeStruct((N_PAD, D), jnp.float32),
    )(p, y1, dis2, b1, W2)


def _tc_final(q, y2, dis2, b2):
    def body(q_ref, y2_ref, d_ref, b_ref, o_ref):
        o_ref[...] = d_ref[...] * (q_ref[0] + q_ref[1] + y2_ref[...]) + b_ref[...]

    return pl.pallas_call(
        body,
        grid=(N_PAD // BLK,),
        in_specs=[
            pl.BlockSpec((NC, BLK, D), lambda i: (0, i, 0)),
            pl.BlockSpec((BLK, D), lambda i: (i, 0)),
            pl.BlockSpec((BLK, 1), lambda i: (i, 0)),
            pl.BlockSpec((1, D), lambda i: (0, 0)),
        ],
        out_specs=pl.BlockSpec((BLK, D), lambda i: (i, 0)),
        out_shape=jax.ShapeDtypeStruct((N_PAD, D), jnp.float32),
    )(q, y2, dis2, b2)


def kernel(x, edge_index, W1, b1, W2, b2):
    N, _ = x.shape
    E = edge_index.shape[1]
    src = edge_index[0].astype(jnp.int32)
    dst = edge_index[1].astype(jnp.int32)

    epw_raw = -(-E // NW)
    chunks = -(-epw_raw // CH)
    chunks += chunks % 2
    epw = chunks * CH
    pad = epw * NW - E
    dum = N + (jnp.arange(pad, dtype=jnp.int32) % (N_PAD - N))
    srcp = jnp.concatenate([src, dum]).reshape(NW, chunks, CH)
    dstp = jnp.concatenate([dst, dum]).reshape(NW, chunks, CH)
    xp = jnp.pad(x, ((0, N_PAD - N), (0, 0)))
    zeros = jnp.zeros((N_PAD, D), jnp.float32)

    degp = _make_deg_kernel(epw)(dstp.reshape(NW, epw))
    deg = degp[0] + degp[1] + 1.0
    dis2 = lax.rsqrt(deg)[:, None]

    agg = _make_agg_kernel(chunks)
    y1 = _tc_scale_matmul(xp, W1, dis2)
    src1d = srcp.reshape(NW, epw)
    p = agg(y1, src1d, dstp, zeros)
    y2 = _tc_mid(p, y1, dis2, b1.reshape(1, D), W2, N)
    q = agg(y2, src1d, dstp, zeros)
    out = _tc_final(q, y2, dis2, b2.reshape(1, D))
    return out[:N]
